# Initial kernel scaffold; baseline (speedup 1.0000x reference)
import jax
import jax.numpy as jnp
from jax.experimental import pallas as pl


def kernel(x, edge_index, mask, dnn_W1, dnn_b1, dnn_W2, dnn_b2, conv1_W, conv1_b, conv2_W, conv2_b, p1_W, p1_b, p2_W, p2_b, fc_W, fc_b):
    raise NotImplementedError("write your pallas kernel here")


# speedup vs baseline: 15.0811x; 15.0811x over previous
import functools

import jax
import jax.numpy as jnp
from jax import lax
from jax.experimental import pallas as pl
from jax.experimental.pallas import tpu as pltpu
from jax.experimental.pallas import tpu_sc as plsc

N = 10000
E = 320000
NPAD = 10240
IN128 = 128

NC = 2
NS = 16
NW = NC * NS
ROWS_PER_TILE = NPAD // NW
EDGES_PER_SC = E // NC
EDGES_PER_TILE = EDGES_PER_SC // NS
EB = 128
N_FULL_BLOCKS = EDGES_PER_TILE // EB
TAIL = EDGES_PER_TILE - N_FULL_BLOCKS * EB
HIST_PER_TILE = NPAD // NS

_mesh = plsc.VectorSubcoreMesh(core_axis_name="c", subcore_axis_name="s")
_sc_params = pltpu.CompilerParams(needs_layout_passes=False)


def _dnn_body(x_ref, m_ref, w1_ref, b1_ref, w2_ref, b2_ref, p1w_ref, p1b_ref,
              p2w_ref, p2b_ref, y_ref, pr1_ref, pr2_ref):
    x = x_ref[...]
    h = jnp.maximum(jnp.dot(x, w1_ref[...],
                            preferred_element_type=jnp.float32) + b1_ref[...], 0.0)
    goid = jnp.maximum(jnp.dot(h, w2_ref[...],
                               preferred_element_type=jnp.float32) + b2_ref[...], 0.0)
    y_ref[...] = jnp.where(m_ref[...] > 0, x[:, :IN128], goid)
    pr1 = jnp.dot(goid, p1w_ref[...],
                  preferred_element_type=jnp.float32) + p1b_ref[...]
    pr1_ref[...] = pr1
    pr2_ref[...] = jnp.dot(pr1, p2w_ref[...],
                           preferred_element_type=jnp.float32) + p2b_ref[...]


def _run_dnn(xp, mask_col, dnn_W1, dnn_b1, dnn_W2, dnn_b2, p1_W, p1_b, p2_W, p2_b):
    blk = 1024
    grid = (NPAD // blk,)
    full = lambda shape: pl.BlockSpec(shape, lambda i: (0,) * len(shape))
    row = lambda w: pl.BlockSpec((blk, w), lambda i: (i, 0))
    return pl.pallas_call(
        _dnn_body,
        grid=grid,
        in_specs=[row(512), row(1),
                  full((512, 1024)), full((1, 1024)),
                  full((1024, 128)), full((1, 128)),
                  full((128, 128)), full((1, 128)),
                  full((128, 128)), full((1, 128))],
        out_specs=[row(128), row(128), row(128)],
        out_shape=[jax.ShapeDtypeStruct((NPAD, 128), jnp.float32)] * 3,
    )(xp, mask_col, dnn_W1, dnn_b1.reshape(1, -1), dnn_W2, dnn_b2.reshape(1, -1),
      p1_W, p1_b.reshape(1, -1), p2_W, p2_b.reshape(1, -1))


def _pos_body(mf_ref, pos_ref):
    m = mf_ref[...]
    a = lax.broadcasted_iota(jnp.int32, (128, 128), 0)
    b = lax.broadcasted_iota(jnp.int32, (128, 128), 1)
    upper = (a <= b).astype(jnp.float32)
    rowcum = jnp.dot(m, upper, preferred_element_type=jnp.float32)
    rowsum = rowcum[:, 127:128]
    r0 = lax.broadcasted_iota(jnp.int32, (80, 80), 0)
    r1 = lax.broadcasted_iota(jnp.int32, (80, 80), 1)
    strict = (r1 < r0).astype(jnp.float32)
    bp = jnp.dot(strict, rowsum, preferred_element_type=jnp.float32)
    incl = rowcum + bp
    total = jnp.sum(m)
    jr = (lax.broadcasted_iota(jnp.int32, (80, 128), 0) * 128 +
          lax.broadcasted_iota(jnp.int32, (80, 128), 1)).astype(jnp.float32)
    posf = jnp.where(m > 0, incl - 1.0, total + jr - incl)
    pos_ref[...] = posf.astype(jnp.int32)


def _run_pos(mask_f):
    return pl.pallas_call(
        _pos_body,
        out_shape=jax.ShapeDtypeStruct((80, 128), jnp.int32),
    )(mask_f)


@functools.partial(
    pl.kernel,
    mesh=_mesh,
    out_type=[jax.ShapeDtypeStruct((NPAD, 128), jnp.float32),
              jax.ShapeDtypeStruct((NC, NPAD), jnp.int32)],
    scratch_types=[
        pltpu.VMEM((EB, 128), jnp.float32),
        pltpu.VMEM((64, 128), jnp.float32),
        pltpu.VMEM((EB,), jnp.int32),
        pltpu.VMEM((64,), jnp.int32),
        pltpu.VMEM((EDGES_PER_TILE,), jnp.int32),
        pltpu.VMEM((NPAD,), jnp.int32),
        pltpu.VMEM((HIST_PER_TILE,), jnp.int32),
        pltpu.VMEM((HIST_PER_TILE,), jnp.int32),
        pltpu.VMEM_SHARED((NS, NPAD), jnp.int32),
        pltpu.SemaphoreType.DMA,
    ],
    compiler_params=_sc_params,
)
def _scatter_hist_kernel(y_hbm, pos_hbm, dst_hbm, xt_hbm, histp_hbm,
                         rows_v, rows64_v, idx_v, idx64_v, dst_v, hist_v,
                         acc_v, tmp_v, hist_sh, sem):
    c = lax.axis_index("c")
    s = lax.axis_index("s")
    w = c * NS + s

    base = w * ROWS_PER_TILE
    for k in range(2):
        off = base + k * EB
        pltpu.sync_copy(y_hbm.at[pl.ds(off, EB)], rows_v)
        pltpu.sync_copy(pos_hbm.at[pl.ds(off, EB)], idx_v)
        pltpu.async_copy(rows_v, xt_hbm.at[idx_v], sem).wait()
    off = base + 2 * EB
    pltpu.sync_copy(y_hbm.at[pl.ds(off, 64)], rows64_v)
    pltpu.sync_copy(pos_hbm.at[pl.ds(off, 64)], idx64_v)
    pltpu.async_copy(rows64_v, xt_hbm.at[idx64_v], sem).wait()

    def zero_hist(i, _):
        hist_v[pl.ds(i * 16, 16)] = jnp.zeros((16,), jnp.int32)
        return _
    lax.fori_loop(0, NPAD // 16, zero_hist, 0)

    pltpu.sync_copy(dst_hbm.at[pl.ds(w * EDGES_PER_TILE, EDGES_PER_TILE)], dst_v)
    ones = jnp.ones((16,), jnp.int32)

    def hist_step(e, _):
        idx = dst_v[pl.ds(e * 16, 16)]
        plsc.addupdate_scatter(hist_v, [idx], ones)
        return _
    lax.fori_loop(0, EDGES_PER_TILE // 16, hist_step, 0)

    pltpu.sync_copy(hist_v, hist_sh.at[s])
    plsc.subcore_barrier()

    hbase = s * HIST_PER_TILE
    pltpu.sync_copy(hist_sh.at[0, pl.ds(hbase, HIST_PER_TILE)], acc_v)
    for t in range(1, NS):
        pltpu.sync_copy(hist_sh.at[t, pl.ds(hbase, HIST_PER_TILE)], tmp_v)

        def add_chunk(i, _, _t=t):
            acc_v[pl.ds(i * 16, 16)] = acc_v[pl.ds(i * 16, 16)] + tmp_v[pl.ds(i * 16, 16)]
            return _
        lax.fori_loop(0, HIST_PER_TILE // 16, add_chunk, 0)
    pltpu.sync_copy(acc_v, histp_hbm.at[c, pl.ds(hbase, HIST_PER_TILE)])


def _xws_body(xt_ref, hist_ref, w_ref, xws_ref, dinv_ref):
    deg = jnp.sum(hist_ref[...].astype(jnp.float32), axis=1, keepdims=True) + 1.0
    dinv = lax.rsqrt(deg)
    z = jnp.dot(xt_ref[...], w_ref[...], preferred_element_type=jnp.float32)
    xws_ref[...] = dinv * z
    dinv_ref[...] = dinv


def _run_xws(xt, hist2, conv_W):
    blk = 1024
    return pl.pallas_call(
        _xws_body,
        grid=(NPAD // blk,),
        in_specs=[pl.BlockSpec((blk, 128), lambda i: (i, 0)),
                  pl.BlockSpec((blk, 2), lambda i: (i, 0)),
                  pl.BlockSpec((128, 128), lambda i: (0, 0))],
        out_specs=[pl.BlockSpec((blk, 128), lambda i: (i, 0)),
                   pl.BlockSpec((blk, 1), lambda i: (i, 0))],
        out_shape=[jax.ShapeDtypeStruct((NPAD, 128), jnp.float32),
                   jax.ShapeDtypeStruct((NPAD, 1), jnp.float32)],
    )(xt, hist2, conv_W)


@functools.partial(
    pl.kernel,
    mesh=_mesh,
    out_type=jax.ShapeDtypeStruct((NC, NPAD, 128), jnp.float32),
    scratch_types=[
        pltpu.VMEM((EB,), jnp.int32),
        pltpu.VMEM((EB,), jnp.int32),
        pltpu.VMEM((EB, 128), jnp.float32),
        pltpu.VMEM((TAIL,), jnp.int32),
        pltpu.VMEM((TAIL,), jnp.int32),
        pltpu.VMEM((TAIL, 128), jnp.float32),
        pltpu.VMEM((EB, 128), jnp.float32),
        pltpu.VMEM_SHARED((NPAD, 128), jnp.float32),
        pltpu.SemaphoreType.DMA,
        pltpu.SemaphoreType.DMA,
    ],
    compiler_params=_sc_params,
)
def _edge_pass_kernel(xws_hbm, src_hbm, dst_hbm, part_hbm,
                      sidx_v, didx_v, rows_v, sidx_t, didx_t, rows_t,
                      zero_v, acc_sh, gsem, ssem):
    c = lax.axis_index("c")
    s = lax.axis_index("s")

    def zero_row(i, _):
        r = i // 8
        k = i % 8
        zero_v[r, pl.ds(k * 16, 16)] = jnp.zeros((16,), jnp.float32)
        return _
    lax.fori_loop(0, EB * 8, zero_row, 0)
    rbase = s * HIST_PER_TILE
    for k in range(HIST_PER_TILE // EB):
        pltpu.sync_copy(zero_v, acc_sh.at[pl.ds(rbase + k * EB, EB)])
    plsc.subcore_barrier()

    ebase = (c * NS + s) * EDGES_PER_TILE

    def block(b, _):
        off = ebase + b * EB
        pltpu.sync_copy(src_hbm.at[pl.ds(off, EB)], sidx_v)
        pltpu.sync_copy(dst_hbm.at[pl.ds(off, EB)], didx_v)
        pltpu.async_copy(xws_hbm.at[sidx_v], rows_v, gsem).wait()
        pltpu.async_copy(rows_v, acc_sh.at[didx_v], ssem, add=True).wait()
        return _
    lax.fori_loop(0, N_FULL_BLOCKS, block, 0)

    toff = ebase + N_FULL_BLOCKS * EB
    pltpu.sync_copy(src_hbm.at[pl.ds(toff, TAIL)], sidx_t)
    pltpu.sync_copy(dst_hbm.at[pl.ds(toff, TAIL)], didx_t)
    pltpu.async_copy(xws_hbm.at[sidx_t], rows_t, gsem).wait()
    pltpu.async_copy(rows_t, acc_sh.at[didx_t], ssem, add=True).wait()

    plsc.subcore_barrier()

    for k in range(HIST_PER_TILE // EB):
        r = rbase + k * EB
        pltpu.sync_copy(acc_sh.at[pl.ds(r, EB)], part_hbm.at[c, pl.ds(r, EB)])


def _mid_body(p0_ref, p1_ref, xws_ref, dinv_ref, pr_ref, m_ref, b_ref, w_ref,
              out_ref):
    agg = p0_ref[...] + p1_ref[...] + xws_ref[...]
    dinv = dinv_ref[...]
    h = jnp.maximum(dinv * agg + b_ref[...], 0.0)
    h = jnp.where(m_ref[...] > 0, h, pr_ref[...])
    out_ref[...] = dinv * jnp.dot(h, w_ref[...],
                                  preferred_element_type=jnp.float32)


def _run_mid(p0, p1, xws, dinv, proj, mask_col, conv_b, next_W):
    blk = 1024
    row = lambda w: pl.BlockSpec((blk, w), lambda i: (i, 0))
    return pl.pallas_call(
        _mid_body,
        grid=(NPAD // blk,),
        in_specs=[row(128), row(128), row(128), row(1), row(128), row(1),
                  pl.BlockSpec((1, 128), lambda i: (0, 0)),
                  pl.BlockSpec((128, 128), lambda i: (0, 0))],
        out_specs=row(128),
        out_shape=jax.ShapeDtypeStruct((NPAD, 128), jnp.float32),
    )(p0, p1, xws, dinv, proj, mask_col, conv_b.reshape(1, -1), next_W)


def _final_body(q0_ref, q1_ref, xws_ref, dinv_ref, pr_ref, m_ref, b_ref,
                fcw_ref, fcb_ref, out_ref):
    agg = q0_ref[...] + q1_ref[...] + xws_ref[...]
    h = jnp.maximum(dinv_ref[...] * agg + b_ref[...], 0.0)
    h = jnp.where(m_ref[...] > 0, h, pr_ref[...])
    out_ref[...] = jnp.dot(h, fcw_ref[...],
                           preferred_element_type=jnp.float32) + fcb_ref[...]


def _run_final(q0, q1, xws, dinv, proj, mask_col, conv_b, fc_W, fc_b):
    blk = 1024
    row = lambda w: pl.BlockSpec((blk, w), lambda i: (i, 0))
    return pl.pallas_call(
        _final_body,
        grid=(NPAD // blk,),
        in_specs=[row(128), row(128), row(128), row(1), row(128), row(1),
                  pl.BlockSpec((1, 128), lambda i: (0, 0)),
                  pl.BlockSpec((128, 64), lambda i: (0, 0)),
                  pl.BlockSpec((1, 64), lambda i: (0, 0))],
        out_specs=row(64),
        out_shape=jax.ShapeDtypeStruct((NPAD, 64), jnp.float32),
    )(q0, q1, xws, dinv, proj, mask_col, conv_b.reshape(1, -1), fc_W,
      fc_b.reshape(1, -1))


def kernel(x, edge_index, mask, dnn_W1, dnn_b1, dnn_W2, dnn_b2,
           conv1_W, conv1_b, conv2_W, conv2_b, p1_W, p1_b, p2_W, p2_b,
           fc_W, fc_b):
    mask = mask.astype(bool)
    xp = jnp.pad(x, ((0, NPAD - N), (0, 0)))
    mask_col = jnp.pad(mask.astype(jnp.int32), (0, NPAD - N)).reshape(NPAD, 1)
    mask_f = jnp.pad(mask.astype(jnp.float32), (0, NPAD - N)).reshape(80, 128)
    src = edge_index[0].astype(jnp.int32)
    dst = edge_index[1].astype(jnp.int32)

    y, proj1, proj2 = _run_dnn(xp, mask_col, dnn_W1, dnn_b1, dnn_W2, dnn_b2,
                               p1_W, p1_b, p2_W, p2_b)
    pos = _run_pos(mask_f).reshape(NPAD)
    xt, histp = _scatter_hist_kernel(y, pos, dst)
    hist2 = histp.reshape(NC, NPAD).transpose(1, 0)

    xws1, dinv = _run_xws(xt, hist2, conv1_W)
    p = _edge_pass_kernel(xws1, src, dst)
    xws2 = _run_mid(p[0], p[1], xws1, dinv, proj1, mask_col, conv1_b, conv2_W)
    q = _edge_pass_kernel(xws2, src, dst)
    out = _run_final(q[0], q[1], xws2, dinv, proj2, mask_col, conv2_b,
                     fc_W, fc_b)
    return out[:N]
